# baseline (speedup 1.0000x reference)
import functools

import jax
import jax.numpy as jnp
from jax import lax
from jax.experimental import pallas as pl
from jax.experimental.pallas import tpu as pltpu
from jax.experimental.pallas import tpu_sc as plsc

V = 8192
N = 32

_NC = 2
_NS = 16


def _gather_body(table_hbm, x_hbm, out_hbm, xv, idxbuf, row, sem):
  c = lax.axis_index("c")
  s = lax.axis_index("s")
  w = c * _NS + s
  pltpu.sync_copy(x_hbm, xv)
  lanes = lax.iota(jnp.int32, 16)
  half = jnp.where(jnp.full((16,), c, jnp.int32) == 0,
                   xv[pl.ds(0, 16)], xv[pl.ds(16, 16)])
  plsc.store_compressed(idxbuf.at[pl.ds(0, 16)], half,
                        mask=lanes == jnp.full((16,), s, jnp.int32))
  pltpu.async_copy(table_hbm.at[idxbuf.at[pl.ds(0, 1)]], row, sem).wait()
  pltpu.sync_copy(row, out_hbm.at[pl.ds(w, 1)])


@functools.lru_cache(maxsize=1)
def _make_gather():
  return pl.kernel(
      _gather_body,
      mesh=plsc.VectorSubcoreMesh(
          core_axis_name="c", subcore_axis_name="s",
          num_cores=_NC, num_subcores=_NS),
      out_type=jax.ShapeDtypeStruct((N, V), jnp.float32),
      compiler_params=pltpu.CompilerParams(needs_layout_passes=False),
      scratch_types=[
          pltpu.VMEM((N,), jnp.int32),
          pltpu.VMEM((16,), jnp.int32),
          pltpu.VMEM((1, V), jnp.float32),
          pltpu.SemaphoreType.DMA,
      ],
  )


def _loss_body(x_sref, y_sref, table_ref, out_ref, buf, sem):
  ids = lax.broadcasted_iota(jnp.int32, (1, V), 1)

  def row_copy(i, slot):
    return pltpu.make_async_copy(
        table_ref.at[pl.ds(x_sref[i], 1)], buf.at[slot], sem.at[slot])

  row_copy(0, 0).start()

  def step(i, acc):
    slot = jnp.bitwise_and(i, 1)
    row_copy(i, slot).wait()

    @pl.when(i < N - 1)
    def _():
      row_copy(i + 1, 1 - slot).start()

    row = buf[slot]
    m = jnp.max(row)
    lse = m + jnp.log(jnp.sum(jnp.exp(row - m)))
    tgt = jnp.sum(jnp.where(ids == y_sref[i], row, 0.0))
    return acc + (lse - tgt)

  acc = lax.fori_loop(0, N, step, jnp.float32(0.0))
  out_ref[0, 0] = acc * (1.0 / N)


@functools.lru_cache(maxsize=1)
def _make_loss():
  return pl.pallas_call(
      _loss_body,
      in_specs=[
          pl.BlockSpec(memory_space=pltpu.SMEM),
          pl.BlockSpec(memory_space=pltpu.SMEM),
          pl.BlockSpec(memory_space=pltpu.HBM),
      ],
      out_specs=pl.BlockSpec(memory_space=pltpu.SMEM),
      scratch_shapes=[
          pltpu.VMEM((2, 1, V), jnp.float32),
          pltpu.SemaphoreType.DMA((2,)),
      ],
      out_shape=jax.ShapeDtypeStruct((1, 1), jnp.float32),
  )


def kernel(x, y, table):
  xf = x.reshape(N).astype(jnp.int32)
  yf = y.reshape(N).astype(jnp.int32)
  logits = _make_gather()(table, xf)
  loss = _make_loss()(xf, yf, table)[0, 0]
  return logits, loss


# speedup vs baseline: 1.8330x; 1.8330x over previous
import functools

import jax
import jax.numpy as jnp
from jax import lax
from jax.experimental import pallas as pl
from jax.experimental.pallas import tpu as pltpu
from jax.experimental.pallas import tpu_sc as plsc

V = 8192
N = 32
H = V // 2

_NC = 2
_NS = 16


def _gather_body(table_hbm, x_hbm, out_hbm, xv, row, sem_in, sem_out):
  c = lax.axis_index("c")
  s = lax.axis_index("s")
  w = c * _NS + s
  pltpu.sync_copy(x_hbm, xv)
  lanes = lax.iota(jnp.int32, 16)
  half = jnp.where(jnp.full((16,), c, jnp.int32) == 0,
                   xv[pl.ds(0, 16)], xv[pl.ds(16, 16)])
  tok = jnp.sum(jnp.where(lanes == jnp.full((16,), s, jnp.int32), half, 0))
  in0 = pltpu.async_copy(table_hbm.at[pl.ds(tok, 1), pl.ds(0, H)],
                         row.at[:, pl.ds(0, H)], sem_in)
  in1 = pltpu.async_copy(table_hbm.at[pl.ds(tok, 1), pl.ds(H, H)],
                         row.at[:, pl.ds(H, H)], sem_in)
  in0.wait()
  out0 = pltpu.async_copy(row.at[:, pl.ds(0, H)],
                          out_hbm.at[pl.ds(w, 1), pl.ds(0, H)], sem_out)
  in1.wait()
  out1 = pltpu.async_copy(row.at[:, pl.ds(H, H)],
                          out_hbm.at[pl.ds(w, 1), pl.ds(H, H)], sem_out)
  out0.wait()
  out1.wait()


@functools.lru_cache(maxsize=1)
def _make_gather():
  return pl.kernel(
      _gather_body,
      mesh=plsc.VectorSubcoreMesh(
          core_axis_name="c", subcore_axis_name="s",
          num_cores=_NC, num_subcores=_NS),
      out_type=jax.ShapeDtypeStruct((N, V), jnp.float32),
      compiler_params=pltpu.CompilerParams(needs_layout_passes=False),
      scratch_types=[
          pltpu.VMEM((N,), jnp.int32),
          pltpu.VMEM((1, V), jnp.float32),
          pltpu.SemaphoreType.DMA,
          pltpu.SemaphoreType.DMA,
      ],
  )


def _loss_body(y_ref, logits_ref, out_ref):
  l = logits_ref[...]
  m = jnp.max(l, axis=1, keepdims=True)
  ssum = jnp.sum(jnp.exp(l - m), axis=1, keepdims=True)
  lse = m + jnp.log(ssum)
  ids = lax.broadcasted_iota(jnp.int32, (N, V), 1)
  tgt = jnp.sum(jnp.where(ids == y_ref[...], l, 0.0), axis=1, keepdims=True)
  out_ref[0, 0] = jnp.sum(lse - tgt) * (1.0 / N)


_loss = pl.pallas_call(
    _loss_body,
    out_shape=jax.ShapeDtypeStruct((1, 1), jnp.float32),
    out_specs=pl.BlockSpec(memory_space=pltpu.SMEM),
)


def kernel(x, y, table):
  xf = x.reshape(N).astype(jnp.int32)
  logits = _make_gather()(table, xf)
  loss = _loss(y.reshape(N, 1).astype(jnp.int32), logits)[0, 0]
  return logits, loss
